# Initial kernel scaffold; baseline (speedup 1.0000x reference)
import functools

import jax
import jax.numpy as jnp
from jax import lax
from jax.experimental import pallas as pl
from jax.experimental.pallas import tpu as pltpu
from jax.experimental.pallas import tpu_sc as plsc

NUM_CORES = 2
NUM_SUBCORES = 16
NW = NUM_CORES * NUM_SUBCORES
LANES = 16

BATCH = 16384
DIM = 64
B_PER_W = BATCH // NW


def _mf_kernel(ids_hbm, vals_hbm, table_hbm, out_hbm,
               idx0_v, idx1_v, rows0_v, rows1_v, v0_v, v1_v, out_v,
               sem0, sem1):
    wid = lax.axis_index("s") * NUM_CORES + lax.axis_index("c")
    base = wid * B_PER_W

    pltpu.sync_copy(ids_hbm.at[0, pl.ds(base, B_PER_W)], idx0_v)
    pltpu.sync_copy(ids_hbm.at[1, pl.ds(base, B_PER_W)], idx1_v)
    pltpu.sync_copy(vals_hbm.at[0, pl.ds(base, B_PER_W)], v0_v)
    pltpu.sync_copy(vals_hbm.at[1, pl.ds(base, B_PER_W)], v1_v)

    cp0 = pltpu.async_copy(table_hbm.at[idx0_v], rows0_v, sem0)
    cp1 = pltpu.async_copy(table_hbm.at[idx1_v], rows1_v, sem1)
    cp0.wait()
    cp1.wait()

    @pl.loop(0, B_PER_W)
    def _(b):
        acc = rows0_v[b, pl.ds(0, LANES)] * rows1_v[b, pl.ds(0, LANES)]
        for d in range(LANES, DIM, LANES):
            acc += rows0_v[b, pl.ds(d, LANES)] * rows1_v[b, pl.ds(d, LANES)]
        out_v[b] = jnp.sum(acc)

    @pl.loop(0, B_PER_W, step=LANES)
    def _(b):
        sl = pl.ds(b, LANES)
        out_v[sl] = out_v[sl] * v0_v[sl] * v1_v[sl]

    pltpu.sync_copy(out_v, out_hbm.at[pl.ds(base, B_PER_W)])


@jax.jit
def kernel(feature_ids, feature_vals, table):
    ids_t = feature_ids.T
    vals_t = feature_vals.T

    mesh = plsc.VectorSubcoreMesh(core_axis_name="c", subcore_axis_name="s")
    run = functools.partial(
        pl.kernel,
        mesh=mesh,
        out_type=jax.ShapeDtypeStruct((BATCH,), jnp.float32),
        scratch_types=[
            pltpu.VMEM((B_PER_W,), jnp.int32),
            pltpu.VMEM((B_PER_W,), jnp.int32),
            pltpu.VMEM((B_PER_W, DIM), jnp.float32),
            pltpu.VMEM((B_PER_W, DIM), jnp.float32),
            pltpu.VMEM((B_PER_W,), jnp.float32),
            pltpu.VMEM((B_PER_W,), jnp.float32),
            pltpu.VMEM((B_PER_W,), jnp.float32),
            pltpu.SemaphoreType.DMA,
            pltpu.SemaphoreType.DMA,
        ],
    )(_mf_kernel)
    return run(ids_t, vals_t, table)


# speedup vs baseline: 2.4628x; 2.4628x over previous
import dataclasses
import functools

import jax
import jax.numpy as jnp
from jax import lax
from jax.experimental import pallas as pl
from jax.experimental.pallas import tpu as pltpu
from jax.experimental.pallas import tpu_sc as plsc

NUM_CORES = 2
NUM_SUBCORES = 16
NW = NUM_CORES * NUM_SUBCORES
LANES = 16

BATCH = 16384
DIM = 64
B_PER_W = BATCH // NW


def _mf_kernel(ids_hbm, vals_hbm, table_hbm, out_hbm,
               idx0_v, idx1_v, rows0_v, rows1_v, v0_v, v1_v, out_v, part_v,
               sem0, sem1):
    wid = lax.axis_index("s") * NUM_CORES + lax.axis_index("c")
    base = wid * B_PER_W

    pltpu.sync_copy(ids_hbm.at[0, pl.ds(base, B_PER_W)], idx0_v)
    pltpu.sync_copy(ids_hbm.at[1, pl.ds(base, B_PER_W)], idx1_v)
    pltpu.sync_copy(vals_hbm.at[0, pl.ds(base, B_PER_W)], v0_v)
    pltpu.sync_copy(vals_hbm.at[1, pl.ds(base, B_PER_W)], v1_v)

    cp0 = pltpu.async_copy(table_hbm.at[idx0_v], rows0_v, sem0)
    cp1 = pltpu.async_copy(table_hbm.at[idx1_v], rows1_v, sem1)
    cp0.wait()
    cp1.wait()

    lane_iota = lax.iota(jnp.int32, LANES)

    @pl.loop(0, B_PER_W, step=LANES)
    def _(g):
        for r in range(LANES):
            b = g + r
            part = rows0_v[b, pl.ds(0, LANES)] * rows1_v[b, pl.ds(0, LANES)]
            for d in range(LANES, DIM, LANES):
                part += rows0_v[b, pl.ds(d, LANES)] * rows1_v[b, pl.ds(d, LANES)]
            part_v[r, pl.ds(0, LANES)] = part
        acc = plsc.load_gather(part_v, [lane_iota, jnp.full((LANES,), 0, jnp.int32)])
        for c in range(1, LANES):
            acc += plsc.load_gather(part_v, [lane_iota, jnp.full((LANES,), c, jnp.int32)])
        sl = pl.ds(g, LANES)
        out_v[sl] = acc * v0_v[sl] * v1_v[sl]

    pltpu.sync_copy(out_v, out_hbm.at[pl.ds(base, B_PER_W)])


@jax.jit
def kernel(feature_ids, feature_vals, table):
    ids_t = feature_ids.T
    vals_t = feature_vals.T

    mesh = plsc.VectorSubcoreMesh(core_axis_name="c", subcore_axis_name="s")
    cp = pltpu.CompilerParams()
    for fld, val in (("needs_layout_passes", False),
                     ("use_tc_tiling_on_sc", False)):
        if fld in pltpu.CompilerParams.__dataclass_fields__:
            cp = dataclasses.replace(cp, **{fld: val})
    run = functools.partial(
        pl.kernel,
        mesh=mesh,
        compiler_params=cp,
        out_type=jax.ShapeDtypeStruct((BATCH,), jnp.float32),
        scratch_types=[
            pltpu.VMEM((B_PER_W,), jnp.int32),
            pltpu.VMEM((B_PER_W,), jnp.int32),
            pltpu.VMEM((B_PER_W, DIM), jnp.float32),
            pltpu.VMEM((B_PER_W, DIM), jnp.float32),
            pltpu.VMEM((B_PER_W,), jnp.float32),
            pltpu.VMEM((B_PER_W,), jnp.float32),
            pltpu.VMEM((B_PER_W,), jnp.float32),
            pltpu.VMEM((LANES, LANES), jnp.float32),
            pltpu.SemaphoreType.DMA,
            pltpu.SemaphoreType.DMA,
        ],
    )(_mf_kernel)
    return run(ids_t, vals_t, table)
